# Initial kernel scaffold; baseline (speedup 1.0000x reference)
import jax
import jax.numpy as jnp
from jax.experimental import pallas as pl


def kernel(tokens, table):
    raise NotImplementedError("write your pallas kernel here")


# speedup vs baseline: 1.7571x; 1.7571x over previous
import functools

import jax
import jax.numpy as jnp
from jax import lax
from jax.experimental import pallas as pl
from jax.experimental.pallas import tpu as pltpu
from jax.experimental.pallas import tpu_sc as plsc

VOCAB = 128512
D = 4096
NC = 2
NS = 16
NW = NC * NS
C = 8
NBUF = 2


@functools.partial(jax.jit, static_argnames=("nch",))
def _embed_sc(tokens_w, table, *, nch):
    mesh = plsc.VectorSubcoreMesh(core_axis_name="c", subcore_axis_name="s")

    @functools.partial(
        pl.kernel,
        mesh=mesh,
        out_type=jax.ShapeDtypeStruct((NW, nch, C, D), jnp.float32),
        scratch_types=[
            pltpu.VMEM((nch, C), jnp.int32),
            pltpu.VMEM((NBUF, C, D), jnp.float32),
            pltpu.SemaphoreType.DMA((NBUF,)),
            pltpu.SemaphoreType.DMA((NBUF,)),
        ],
    )
    def body(tok_hbm, table_hbm, out_hbm, idx_v, rows_v, gsem, wsem):
        wid = lax.axis_index("s") * NC + lax.axis_index("c")

        pltpu.sync_copy(tok_hbm.at[wid], idx_v)

        def start_gather(g, b):
            pltpu.async_copy(table_hbm.at[idx_v.at[g]], rows_v.at[b],
                             gsem.at[b])

        def wait_gather(b):
            pltpu.make_async_copy(table_hbm.at[idx_v.at[0]], rows_v.at[b],
                                  gsem.at[b]).wait()

        def start_write(g, b):
            pltpu.async_copy(rows_v.at[b], out_hbm.at[wid, g], wsem.at[b])

        def wait_write(b):
            pltpu.make_async_copy(rows_v.at[b], out_hbm.at[wid, 0],
                                  wsem.at[b]).wait()

        for b in range(NBUF):
            start_gather(b, b)

        @pl.loop(0, nch, step=NBUF)
        def _(g0):
            for b in range(NBUF):
                g = g0 + b
                wait_gather(b)
                start_write(g, b)
                nxt = g + NBUF

                @pl.when(nxt < nch)
                def _():
                    wait_write(b)
                    start_gather(nxt, b)

        for b in range(NBUF):
            wait_write(b)

    return body(tokens_w, table)


def kernel(tokens, table):
    b, s = tokens.shape
    total = b * s
    nch = total // (NW * C)
    tokens_w = tokens.reshape(NW, nch, C).astype(jnp.int32)
    out = _embed_sc(tokens_w, table, nch=nch)
    return out.reshape(b, s, D)
